# baseline (speedup 1.0000x reference)
import functools

import jax
import jax.numpy as jnp
from jax import lax
from jax.experimental import pallas as pl
from jax.experimental.pallas import tpu as pltpu
from jax.experimental.pallas import tpu_sc as plsc

NUM_CORES = 2
NUM_SUBCORES = 16
NW = NUM_CORES * NUM_SUBCORES

BATCH = 4096
SRC_LEN = 200
EMBED_DIM = 64
TOTAL = BATCH * SRC_LEN
BPW = TOTAL // NW
CHUNK = 512
NBUF = 2
NCHUNK = BPW // CHUNK
NGRP = NCHUNK // NBUF

_mesh = plsc.VectorSubcoreMesh(
    core_axis_name="c", subcore_axis_name="s",
    num_cores=NUM_CORES, num_subcores=NUM_SUBCORES,
)

_row_buf = pltpu.VMEM((CHUNK, EMBED_DIM), jnp.float32)


@functools.partial(
    pl.kernel,
    out_type=jax.ShapeDtypeStruct((TOTAL, EMBED_DIM), jnp.float32),
    mesh=_mesh,
    compiler_params=pltpu.CompilerParams(use_tc_tiling_on_sc=False),
    scratch_types=[
        pltpu.VMEM((BPW,), jnp.int32),
        [_row_buf] * NBUF,
        [pltpu.SemaphoreType.DMA] * NBUF,
        [pltpu.SemaphoreType.DMA] * NBUF,
    ],
)
def _sc_gather(idx_hbm, table_hbm, out_hbm, idx_v, rows, sg, ss):
    wid = lax.axis_index("s") * NUM_CORES + lax.axis_index("c")
    base = wid * BPW
    pltpu.sync_copy(idx_hbm.at[pl.ds(base, BPW)], idx_v)

    def gather_start(g, b):
        pltpu.async_copy(table_hbm.at[idx_v.at[pl.ds(g * CHUNK, CHUNK)]],
                         rows[b], sg[b])

    def store_start(g, b):
        pltpu.async_copy(rows[b], out_hbm.at[pl.ds(base + g * CHUNK, CHUNK)],
                         ss[b])

    def gather_wait(b):
        pltpu.make_async_copy(table_hbm.at[idx_v.at[pl.ds(0, CHUNK)]],
                              rows[b], sg[b]).wait()

    def store_wait(g, b):
        pltpu.make_async_copy(rows[b],
                              out_hbm.at[pl.ds(base + g * CHUNK, CHUNK)],
                              ss[b]).wait()

    for b in range(NBUF):
        gather_start(b, b)

    @pl.loop(0, NGRP - 1)
    def _round(i):
        g0 = i * NBUF
        for b in range(NBUF):
            gather_wait(b)
            store_start(g0 + b, b)
        for b in range(NBUF):
            store_wait(g0 + b, b)
            gather_start(g0 + NBUF + b, b)

    g0 = NCHUNK - NBUF
    for b in range(NBUF):
        gather_wait(b)
        store_start(g0 + b, b)
    for b in range(NBUF):
        store_wait(g0 + b, b)


RCHUNK = 320
RNCHUNK = BPW // RCHUNK
L = 16


@functools.partial(
    pl.kernel,
    out_type=jax.ShapeDtypeStruct((TOTAL, EMBED_DIM), jnp.float32),
    mesh=_mesh,
    compiler_params=pltpu.CompilerParams(use_tc_tiling_on_sc=True),
    scratch_types=[
        [pltpu.VMEM((RCHUNK // 2, 128), jnp.float32)] * 2,
        [pltpu.VMEM((RCHUNK, EMBED_DIM), jnp.float32)] * 2,
        [pltpu.SemaphoreType.DMA] * 2,
        [pltpu.SemaphoreType.DMA] * 2,
    ],
)
def _sc_repack(packed_hbm, out_hbm, ibuf, obuf, si, so):
    wid = lax.axis_index("s") * NUM_CORES + lax.axis_index("c")
    base = wid * BPW

    def in_start(g, b):
        off = pl.multiple_of((base + g * RCHUNK) // 2, 8)
        pltpu.async_copy(packed_hbm.at[pl.ds(off, RCHUNK // 2)],
                         ibuf[b], si[b])

    def in_wait(g, b):
        off = pl.multiple_of((base + g * RCHUNK) // 2, 8)
        pltpu.make_async_copy(packed_hbm.at[pl.ds(off, RCHUNK // 2)],
                              ibuf[b], si[b]).wait()

    def out_start(g, b):
        off = pl.multiple_of(base + g * RCHUNK, 8)
        pltpu.async_copy(obuf[b], out_hbm.at[pl.ds(off, RCHUNK)], so[b])

    def out_wait(g, b):
        off = pl.multiple_of(base + g * RCHUNK, 8)
        pltpu.make_async_copy(obuf[b], out_hbm.at[pl.ds(off, RCHUNK)],
                              so[b]).wait()

    def shuffle(b):
        @pl.loop(0, RCHUNK // 2)
        def _line(q):
            for h in range(2):
                for k in range(EMBED_DIM // L):
                    obuf[b][2 * q + h, pl.ds(k * L, L)] = (
                        ibuf[b][q, pl.ds(h * EMBED_DIM + k * L, L)])

    in_start(0, 0)
    in_start(1, 1)

    @pl.loop(0, RNCHUNK // 2 - 1)
    def _pair(i):
        g0 = 2 * i
        for b in range(2):
            g = g0 + b
            in_wait(g, b)
            shuffle(b)
            out_start(g, b)
            out_wait(g, b)
            in_start(g + 2, b)

    g0 = RNCHUNK - 2
    for b in range(2):
        g = g0 + b
        in_wait(g, b)
        shuffle(b)
        out_start(g, b)
        out_wait(g, b)


def kernel(src, table):
    idx = src.reshape(TOTAL).astype(jnp.int32)
    packed = _sc_gather(idx, table).reshape(TOTAL // 2, 2 * EMBED_DIM)
    out = _sc_repack(packed)
    return out.reshape(BATCH, SRC_LEN, EMBED_DIM)


# speedup vs baseline: 1.0993x; 1.0993x over previous
import functools

import jax
import jax.numpy as jnp
from jax import lax
from jax.experimental import pallas as pl
from jax.experimental.pallas import tpu as pltpu
from jax.experimental.pallas import tpu_sc as plsc

NUM_CORES = 2
NUM_SUBCORES = 16
NW = NUM_CORES * NUM_SUBCORES

BATCH = 4096
SRC_LEN = 200
EMBED_DIM = 64
TOTAL = BATCH * SRC_LEN
BPW = TOTAL // NW
CHUNK = 512
NBUF = 2
NCHUNK = BPW // CHUNK
NGRP = NCHUNK // NBUF

_mesh = plsc.VectorSubcoreMesh(
    core_axis_name="c", subcore_axis_name="s",
    num_cores=NUM_CORES, num_subcores=NUM_SUBCORES,
)

_row_buf = pltpu.VMEM((CHUNK, EMBED_DIM), jnp.float32)


@functools.partial(
    pl.kernel,
    out_type=jax.ShapeDtypeStruct((TOTAL, EMBED_DIM), jnp.float32),
    mesh=_mesh,
    compiler_params=pltpu.CompilerParams(use_tc_tiling_on_sc=False),
    scratch_types=[
        pltpu.VMEM((BPW,), jnp.int32),
        [_row_buf] * NBUF,
        [pltpu.SemaphoreType.DMA] * NBUF,
        [pltpu.SemaphoreType.DMA] * NBUF,
    ],
)
def _sc_gather(idx_hbm, table_hbm, out_hbm, idx_v, rows, sg, ss):
    wid = lax.axis_index("s") * NUM_CORES + lax.axis_index("c")
    base = wid * BPW
    pltpu.sync_copy(idx_hbm.at[pl.ds(base, BPW)], idx_v)

    def gather_start(g, b):
        pltpu.async_copy(table_hbm.at[idx_v.at[pl.ds(g * CHUNK, CHUNK)]],
                         rows[b], sg[b])

    def store_start(g, b):
        pltpu.async_copy(rows[b], out_hbm.at[pl.ds(base + g * CHUNK, CHUNK)],
                         ss[b])

    def gather_wait(b):
        pltpu.make_async_copy(table_hbm.at[idx_v.at[pl.ds(0, CHUNK)]],
                              rows[b], sg[b]).wait()

    def store_wait(g, b):
        pltpu.make_async_copy(rows[b],
                              out_hbm.at[pl.ds(base + g * CHUNK, CHUNK)],
                              ss[b]).wait()

    for b in range(NBUF):
        gather_start(b, b)

    @pl.loop(0, NGRP - 1)
    def _round(i):
        g0 = i * NBUF
        for b in range(NBUF):
            gather_wait(b)
            store_start(g0 + b, b)
        for b in range(NBUF):
            store_wait(g0 + b, b)
            gather_start(g0 + NBUF + b, b)

    g0 = NCHUNK - NBUF
    for b in range(NBUF):
        gather_wait(b)
        store_start(g0 + b, b)
    for b in range(NBUF):
        store_wait(g0 + b, b)


RCHUNK = 320
RNCHUNK = BPW // RCHUNK
L = 16


@functools.partial(
    pl.kernel,
    out_type=jax.ShapeDtypeStruct((TOTAL, EMBED_DIM), jnp.float32),
    mesh=_mesh,
    compiler_params=pltpu.CompilerParams(use_tc_tiling_on_sc=True),
    scratch_types=[
        [pltpu.VMEM((RCHUNK // 2, 128), jnp.float32)] * 2,
        [pltpu.VMEM((RCHUNK, EMBED_DIM), jnp.float32)] * 2,
        [pltpu.SemaphoreType.DMA] * 2,
        [pltpu.SemaphoreType.DMA] * 2,
    ],
)
def _sc_repack(packed_hbm, out_hbm, ibuf, obuf, si, so):
    wid = lax.axis_index("s") * NUM_CORES + lax.axis_index("c")
    base = wid * BPW

    def in_start(g, b):
        off = pl.multiple_of((base + g * RCHUNK) // 2, 8)
        pltpu.async_copy(packed_hbm.at[pl.ds(off, RCHUNK // 2)],
                         ibuf[b], si[b])

    def in_wait(g, b):
        off = pl.multiple_of((base + g * RCHUNK) // 2, 8)
        pltpu.make_async_copy(packed_hbm.at[pl.ds(off, RCHUNK // 2)],
                              ibuf[b], si[b]).wait()

    def out_start(g, b):
        off = pl.multiple_of(base + g * RCHUNK, 8)
        pltpu.async_copy(obuf[b], out_hbm.at[pl.ds(off, RCHUNK)], so[b])

    def out_wait(g, b):
        off = pl.multiple_of(base + g * RCHUNK, 8)
        pltpu.make_async_copy(obuf[b], out_hbm.at[pl.ds(off, RCHUNK)],
                              so[b]).wait()

    def shuffle(b):
        @pl.loop(0, RCHUNK // 2, unroll=8)
        def _line(q):
            for h in range(2):
                for k in range(EMBED_DIM // L):
                    obuf[b][2 * q + h, pl.ds(k * L, L)] = (
                        ibuf[b][q, pl.ds(h * EMBED_DIM + k * L, L)])

    in_start(0, 0)
    in_start(1, 1)
    for b in range(2):
        in_wait(b, b)
        shuffle(b)
        out_start(b, b)
        in_start(b + 2, b)

    @pl.loop(1, RNCHUNK // 2 - 1)
    def _pair(i):
        g0 = 2 * i
        for b in range(2):
            g = g0 + b
            in_wait(g, b)
            out_wait(g - 2, b)
            shuffle(b)
            out_start(g, b)
            in_start(g + 2, b)

    g0 = RNCHUNK - 2
    for b in range(2):
        g = g0 + b
        in_wait(g, b)
        out_wait(g - 2, b)
        shuffle(b)
        out_start(g, b)
    for b in range(2):
        out_wait(g0 + b, b)


def kernel(src, table):
    idx = src.reshape(TOTAL).astype(jnp.int32)
    packed = _sc_gather(idx, table).reshape(TOTAL // 2, 2 * EMBED_DIM)
    out = _sc_repack(packed)
    return out.reshape(BATCH, SRC_LEN, EMBED_DIM)


# speedup vs baseline: 1.1460x; 1.0424x over previous
import functools

import jax
import jax.numpy as jnp
from jax import lax
from jax.experimental import pallas as pl
from jax.experimental.pallas import tpu as pltpu
from jax.experimental.pallas import tpu_sc as plsc

NUM_CORES = 2
NUM_SUBCORES = 16
NW = NUM_CORES * NUM_SUBCORES

BATCH = 4096
SRC_LEN = 200
EMBED_DIM = 64
TOTAL = BATCH * SRC_LEN
BPW = TOTAL // NW
CHUNK = 512
NBUF = 2
NCHUNK = BPW // CHUNK
NGRP = NCHUNK // NBUF

_mesh = plsc.VectorSubcoreMesh(
    core_axis_name="c", subcore_axis_name="s",
    num_cores=NUM_CORES, num_subcores=NUM_SUBCORES,
)

_row_buf = pltpu.VMEM((CHUNK, EMBED_DIM), jnp.float32)


@functools.partial(
    pl.kernel,
    out_type=jax.ShapeDtypeStruct((TOTAL, EMBED_DIM), jnp.float32),
    mesh=_mesh,
    compiler_params=pltpu.CompilerParams(use_tc_tiling_on_sc=False),
    scratch_types=[
        pltpu.VMEM((BPW,), jnp.int32),
        [_row_buf] * NBUF,
        [pltpu.SemaphoreType.DMA] * NBUF,
        [pltpu.SemaphoreType.DMA] * NBUF,
    ],
)
def _sc_gather(idx_hbm, table_hbm, out_hbm, idx_v, rows, sg, ss):
    wid = lax.axis_index("s") * NUM_CORES + lax.axis_index("c")
    base = wid * BPW
    pltpu.sync_copy(idx_hbm.at[pl.ds(base, BPW)], idx_v)

    def gather_start(g, b):
        pltpu.async_copy(table_hbm.at[idx_v.at[pl.ds(g * CHUNK, CHUNK)]],
                         rows[b], sg[b])

    def store_start(g, b):
        pltpu.async_copy(rows[b], out_hbm.at[pl.ds(base + g * CHUNK, CHUNK)],
                         ss[b])

    def gather_wait(b):
        pltpu.make_async_copy(table_hbm.at[idx_v.at[pl.ds(0, CHUNK)]],
                              rows[b], sg[b]).wait()

    def store_wait(g, b):
        pltpu.make_async_copy(rows[b],
                              out_hbm.at[pl.ds(base + g * CHUNK, CHUNK)],
                              ss[b]).wait()

    for b in range(NBUF):
        gather_start(b, b)

    @pl.loop(0, NGRP - 1)
    def _round(i):
        g0 = i * NBUF
        for b in range(NBUF):
            gather_wait(b)
            store_start(g0 + b, b)
        for b in range(NBUF):
            store_wait(g0 + b, b)
            gather_start(g0 + NBUF + b, b)

    g0 = NCHUNK - NBUF
    for b in range(NBUF):
        gather_wait(b)
        store_start(g0 + b, b)
    for b in range(NBUF):
        store_wait(g0 + b, b)


def kernel(src, table):
    idx = src.reshape(TOTAL).astype(jnp.int32)
    out = _sc_gather(idx, table)
    return out.reshape(BATCH, SRC_LEN, EMBED_DIM)
